# Initial kernel scaffold; baseline (speedup 1.0000x reference)
import jax
import jax.numpy as jnp
from jax.experimental import pallas as pl


def kernel(z_e, codebook):
    raise NotImplementedError("write your pallas kernel here")


# speedup vs baseline: 1.1823x; 1.1823x over previous
import jax
import jax.numpy as jnp
from jax.experimental import pallas as pl

_NUM_EMB = 1024
_EMB_DIM = 256
_COMMIT = 0.25
_TS = 512


def _vq_tile(z_ref, cb_ref, zq_ref, idx_ref, loss_ref):
    zb = z_ref[0]
    cb = cb_ref[...]
    cb_sq = jnp.sum(cb * cb, axis=1)
    z_sq = jnp.sum(zb * zb, axis=0)

    dot = jax.lax.dot_general(
        zb, cb, (((0,), (1,)), ((), ())),
        preferred_element_type=jnp.float32)
    scores = (z_sq[:, None] + cb_sq[None, :]) - 2.0 * dot

    min_val = jnp.min(scores, axis=1, keepdims=True)
    lane = jax.lax.broadcasted_iota(jnp.int32, scores.shape, 1)
    idx = jnp.min(jnp.where(scores == min_val, lane, _NUM_EMB),
                  axis=1)

    oh = (lane == idx[:, None]).astype(jnp.bfloat16)
    cb_hi = cb.astype(jnp.bfloat16)
    cb_lo = (cb - cb_hi.astype(jnp.float32)).astype(jnp.bfloat16)
    dn = (((0,), (1,)), ((), ()))
    zq = (jax.lax.dot_general(cb_hi, oh, dn,
                              preferred_element_type=jnp.float32)
          + jax.lax.dot_general(cb_lo, oh, dn,
                                preferred_element_type=jnp.float32))

    zq_ref[0] = zb + (zq - zb)
    idx_ref[0, 0] = idx

    part = jnp.sum(min_val, axis=0, keepdims=True)

    @pl.when((pl.program_id(0) == 0) & (pl.program_id(1) == 0))
    def _():
        loss_ref[...] = jnp.zeros_like(loss_ref)

    loss_ref[...] += part


def kernel(z_e, codebook):
    B, C, D, H, W = z_e.shape
    S = D * H * W
    z = z_e.reshape(B, C, S)

    zq, idx, loss = pl.pallas_call(
        _vq_tile,
        grid=(B, S // _TS),
        in_specs=[
            pl.BlockSpec((1, C, _TS), lambda b, t: (b, 0, t)),
            pl.BlockSpec((_NUM_EMB, _EMB_DIM), lambda b, t: (0, 0)),
        ],
        out_specs=[
            pl.BlockSpec((1, C, _TS), lambda b, t: (b, 0, t)),
            pl.BlockSpec((1, 1, _TS), lambda b, t: (b, 0, t)),
            pl.BlockSpec((1, 1), lambda b, t: (0, 0)),
        ],
        out_shape=[
            jax.ShapeDtypeStruct((B, C, S), jnp.float32),
            jax.ShapeDtypeStruct((B, 1, S), jnp.int32),
            jax.ShapeDtypeStruct((1, 1), jnp.float32),
        ],
    )(z, codebook)

    m = loss[0, 0] / jnp.float32(B * S * C)
    vq_loss = m + jnp.float32(_COMMIT) * m
    return (zq.reshape(B, C, D, H, W), vq_loss,
            idx.reshape(B, D, H, W))


# speedup vs baseline: 1.2624x; 1.0678x over previous
import jax
import jax.numpy as jnp
from jax.experimental import pallas as pl
from jax.experimental.pallas import tpu as pltpu

_NUM_EMB = 1024
_EMB_DIM = 256
_COMMIT = 0.25
_TS = 512


def _vq_tile(z_ref, cb2_ref, cbt_hi_ref, cbt_lo_ref,
             zq_ref, idx_ref, loss_ref, csq_ref):
    first = (pl.program_id(0) == 0) & (pl.program_id(1) == 0)

    @pl.when(first)
    def _():
        cb2 = cb2_ref[...]
        csq = 0.25 * jnp.sum(cb2 * cb2, axis=1, keepdims=True)
        csq_ref[...] = jnp.broadcast_to(csq, (_NUM_EMB, _TS))
        loss_ref[...] = jnp.zeros_like(loss_ref)

    zb = z_ref[0]
    dot2 = jax.lax.dot_general(
        cb2_ref[...], zb, (((1,), (0,)), ((), ())),
        preferred_element_type=jnp.float32)
    z_sq = jnp.sum(zb * zb, axis=0, keepdims=True)
    scores = (csq_ref[...] + z_sq) + dot2

    minv = jnp.min(scores, axis=0, keepdims=True)
    row = jax.lax.broadcasted_iota(jnp.int32, scores.shape, 0)
    idx = jnp.min(jnp.where(scores == minv, row, _NUM_EMB),
                  axis=0, keepdims=True)

    oh = (row == idx).astype(jnp.bfloat16)
    dn = (((1,), (0,)), ((), ()))
    zq = (jax.lax.dot_general(cbt_hi_ref[...], oh, dn,
                              preferred_element_type=jnp.float32)
          + jax.lax.dot_general(cbt_lo_ref[...], oh, dn,
                                preferred_element_type=jnp.float32))

    zq_ref[0] = zb + (zq - zb)
    idx_ref[0] = idx
    loss_ref[...] += jnp.sum(minv, axis=1, keepdims=True)


def kernel(z_e, codebook):
    B, C, D, H, W = z_e.shape
    S = D * H * W
    z = z_e.reshape(B, C, S)

    cb2 = -2.0 * codebook
    cbt = codebook.T
    cbt_hi = cbt.astype(jnp.bfloat16)
    cbt_lo = (cbt - cbt_hi.astype(jnp.float32)).astype(jnp.bfloat16)

    zq, idx, loss = pl.pallas_call(
        _vq_tile,
        grid=(B, S // _TS),
        in_specs=[
            pl.BlockSpec((1, C, _TS), lambda b, t: (b, 0, t)),
            pl.BlockSpec((_NUM_EMB, _EMB_DIM), lambda b, t: (0, 0)),
            pl.BlockSpec((_EMB_DIM, _NUM_EMB), lambda b, t: (0, 0)),
            pl.BlockSpec((_EMB_DIM, _NUM_EMB), lambda b, t: (0, 0)),
        ],
        out_specs=[
            pl.BlockSpec((1, C, _TS), lambda b, t: (b, 0, t)),
            pl.BlockSpec((1, 1, _TS), lambda b, t: (b, 0, t)),
            pl.BlockSpec((1, 1), lambda b, t: (0, 0)),
        ],
        out_shape=[
            jax.ShapeDtypeStruct((B, C, S), jnp.float32),
            jax.ShapeDtypeStruct((B, 1, S), jnp.int32),
            jax.ShapeDtypeStruct((1, 1), jnp.float32),
        ],
        scratch_shapes=[pltpu.VMEM((_NUM_EMB, _TS), jnp.float32)],
    )(z, cb2, cbt_hi, cbt_lo)

    m = loss[0, 0] / jnp.float32(B * S * C)
    vq_loss = m + jnp.float32(_COMMIT) * m
    return (zq.reshape(B, C, D, H, W), vq_loss,
            idx.reshape(B, D, H, W))
